# Initial kernel scaffold; baseline (speedup 1.0000x reference)
import jax
import jax.numpy as jnp
from jax.experimental import pallas as pl


def kernel(x, mask, position_embedding):
    raise NotImplementedError("write your pallas kernel here")


# speedup vs baseline: 2.6001x; 2.6001x over previous
import jax
import jax.numpy as jnp
from jax import lax
from jax.experimental import pallas as pl
from jax.experimental.pallas import tpu as pltpu
from jax.experimental.pallas import tpu_sc as plsc

_NC, _NS = 2, 16
_NW = _NC * _NS
_B, _S, _D = 4096, 200, 64
_ROWS_PER_W = _B // _NW
_XLEN = _S * _D
_TABLE_ROWS = 101
_TLEN = _TABLE_ROWS * _D
_MPAD = 208
_C = 2
_NB = 4
_NCH = _ROWS_PER_W // _C


def _body(x_hbm, m_hbm, t_hbm, out_hbm, tabv, descv,
          mbuf0, mbuf1, mbuf2, mbuf3, obuf0, obuf1, obuf2, obuf3,
          xin_sem, min_sem, out_sem):
    mbufs = [mbuf0, mbuf1, mbuf2, mbuf3]
    obufs = [obuf0, obuf1, obuf2, obuf3]
    wid = lax.axis_index("s") * _NC + lax.axis_index("c")
    row0 = wid * _ROWS_PER_W

    pltpu.sync_copy(t_hbm, tabv)

    def _rev(j, carry):
        for g in range(4):
            descv[pl.ds(j * _D + g * 16, 16)] = tabv[pl.ds((100 - j) * _D + g * 16, 16)]
        return carry

    lax.fori_loop(0, 100, _rev, 0)

    def _start_in(c, b):
        r = row0 + c * _C
        pltpu.async_copy(x_hbm.at[pl.ds(r * _XLEN, _C * _XLEN)], obufs[b],
                         xin_sem.at[b])
        pltpu.async_copy(m_hbm.at[pl.ds(r * _MPAD, _C * _MPAD)], mbufs[b],
                         min_sem.at[b])

    def _wait_out(b):
        pltpu.make_async_copy(obufs[b], out_hbm.at[pl.ds(0, _C * _XLEN)],
                              out_sem.at[b]).wait()

    def _chunk(c, b):
        pltpu.make_async_copy(x_hbm.at[pl.ds(0, _C * _XLEN)], obufs[b],
                              xin_sem.at[b]).wait()
        pltpu.make_async_copy(m_hbm.at[pl.ds(0, _C * _MPAD)], mbufs[b],
                              min_sem.at[b]).wait()
        for r in range(_C):
            mb = r * _MPAD
            acc = mbufs[b][pl.ds(mb, 16)]
            for cc in range(1, _MPAD // 16):
                acc = acc + mbufs[b][pl.ds(mb + cc * 16, 16)]
            vl = acc[0]
            for l in range(1, 16):
                vl = vl + acc[l]
            h2 = vl // 2
            h1 = vl - h2
            xb = r * _XLEN

            def _asc(k, c2):
                for g in range(4):
                    plsc.addupdate(obufs[b].at[pl.ds(xb + k * _D + g * 16, 16)],
                                   tabv[pl.ds(_D + k * _D + g * 16, 16)])
                return c2

            lax.fori_loop(0, h1, _asc, 0)

            off = xb + h1 * _D
            doff = (100 - h2) * _D

            def _desc(k, c2):
                for g in range(4):
                    plsc.addupdate(obufs[b].at[pl.ds(off + k * _D + g * 16, 16)],
                                   descv[pl.ds(doff + k * _D + g * 16, 16)])
                return c2

            lax.fori_loop(0, h2, _desc, 0)
        r = row0 + c * _C
        pltpu.async_copy(obufs[b], out_hbm.at[pl.ds(r * _XLEN, _C * _XLEN)],
                         out_sem.at[b])

    _start_in(0, 0)
    _start_in(1, 1)

    @pl.loop(0, _NCH, step=_NB)
    def _outer(g):
        for b in range(_NB):
            c = g + b
            nc = c + 2
            bb = (b + 2) % _NB
            if b < 2:
                @pl.when(g > 0)
                def _w():
                    _wait_out(bb)
                    _start_in(nc, bb)

                @pl.when(g == 0)
                def _s():
                    _start_in(nc, bb)
            else:
                @pl.when(g < _NCH - _NB)
                def _ws():
                    _wait_out(bb)
                    _start_in(nc, bb)
            _chunk(c, b)

    for b in range(_NB):
        _wait_out(b)


def kernel(x, mask, position_embedding):
    b, s, d = x.shape
    xf = x.reshape(-1)
    mi = jnp.pad(mask.astype(jnp.int32), ((0, 0), (0, _MPAD - s))).reshape(-1)
    tf = position_embedding.reshape(-1)
    mesh = plsc.VectorSubcoreMesh(
        core_axis_name="c", subcore_axis_name="s", num_cores=_NC, num_subcores=_NS
    )
    out = pl.kernel(
        _body,
        out_type=jax.ShapeDtypeStruct((b * s * d,), jnp.float32),
        mesh=mesh,
        scratch_types=[
            pltpu.VMEM((_TLEN,), jnp.float32),
            pltpu.VMEM((100 * _D,), jnp.float32),
            pltpu.VMEM((_C * _MPAD,), jnp.int32),
            pltpu.VMEM((_C * _MPAD,), jnp.int32),
            pltpu.VMEM((_C * _MPAD,), jnp.int32),
            pltpu.VMEM((_C * _MPAD,), jnp.int32),
            pltpu.VMEM((_C * _XLEN,), jnp.float32),
            pltpu.VMEM((_C * _XLEN,), jnp.float32),
            pltpu.VMEM((_C * _XLEN,), jnp.float32),
            pltpu.VMEM((_C * _XLEN,), jnp.float32),
            pltpu.SemaphoreType.DMA((_NB,)),
            pltpu.SemaphoreType.DMA((_NB,)),
            pltpu.SemaphoreType.DMA((_NB,)),
        ],
    )(xf, mi, tf)
    return out.reshape(b, s, d)
